# Initial kernel scaffold; baseline (speedup 1.0000x reference)
import jax
import jax.numpy as jnp
from jax.experimental import pallas as pl


def kernel(x, batch, dim, Ww1, bw1, Ww2, bw2, Wg1, bg1, Wg2, bg2, Wm1, bm1, Wm2, bm2):
    raise NotImplementedError("write your pallas kernel here")


# speedup vs baseline: 5.6248x; 5.6248x over previous
import functools

import jax
import jax.numpy as jnp
from jax.experimental import pallas as pl
from jax.experimental.pallas import tpu as pltpu

N = 100000
D = 128
G = 256
BLK = 2000


def _fused_body(x_ref, b_ref, Ww1, bw1, Ww2, bw2, Wg1, bg1, Wg2, bg2,
                Wm1, bm1, Wm2, bm2, out_ref, acc_ref):
    i = pl.program_id(0)
    x = x_ref[...]
    w = jnp.maximum(jax.lax.dot(x, Ww1[...], preferred_element_type=jnp.float32)
                    + bw1[...], 0.0)
    w = jax.lax.dot(w, Ww2[...], preferred_element_type=jnp.float32) + bw2[...]
    g = jnp.maximum(jax.lax.dot(x, Wg1[...], preferred_element_type=jnp.float32)
                    + bg1[...], 0.0)
    g = jax.lax.dot(g, Wg2[...], preferred_element_type=jnp.float32) + bg2[...]
    h = jax.nn.sigmoid(g) * w

    ids = b_ref[0, 0, :]
    onehot = (jax.lax.broadcasted_iota(jnp.int32, (G, BLK), 0)
              == ids[None, :]).astype(jnp.float32)
    part = jax.lax.dot(onehot, h, preferred_element_type=jnp.float32)

    @pl.when(i == 0)
    def _init():
        acc_ref[...] = part

    @pl.when(i > 0)
    def _accum():
        acc_ref[...] += part

    @pl.when(i == pl.num_programs(0) - 1)
    def _final():
        acc = acc_ref[...]
        m = jnp.maximum(jax.lax.dot(acc, Wm1[...], preferred_element_type=jnp.float32)
                        + bm1[...], 0.0)
        out_ref[...] = (jax.lax.dot(m, Wm2[...], preferred_element_type=jnp.float32)
                        + bm2[...])


@functools.partial(jax.jit, static_argnums=(2,))
def _run(x, batch_i32, nblk, Ww1, bw1, Ww2, bw2, Wg1, bg1, Wg2, bg2,
         Wm1, bm1, Wm2, bm2):
    b3 = batch_i32.reshape(nblk, 1, BLK)
    row_spec = pl.BlockSpec((BLK, D), lambda i: (i, 0))
    id_spec = pl.BlockSpec((1, 1, BLK), lambda i: (i, 0, 0))
    w_spec = pl.BlockSpec((D, D), lambda i: (0, 0))
    b_spec = pl.BlockSpec((1, D), lambda i: (0, 0))
    out_spec = pl.BlockSpec((G, D), lambda i: (0, 0))
    return pl.pallas_call(
        _fused_body,
        grid=(nblk,),
        in_specs=[row_spec, id_spec] + [w_spec, b_spec] * 6,
        out_specs=out_spec,
        out_shape=jax.ShapeDtypeStruct((G, D), jnp.float32),
        scratch_shapes=[pltpu.VMEM((G, D), jnp.float32)],
    )(x, b3, Ww1, bw1.reshape(1, D), Ww2, bw2.reshape(1, D),
      Wg1, bg1.reshape(1, D), Wg2, bg2.reshape(1, D),
      Wm1, bm1.reshape(1, D), Wm2, bm2.reshape(1, D))


def kernel(x, batch, dim, Ww1, bw1, Ww2, bw2, Wg1, bg1, Wg2, bg2,
           Wm1, bm1, Wm2, bm2):
    del dim
    batch_i32 = batch.astype(jnp.int32)
    assert x.shape == (N, D) and N % BLK == 0
    return _run(x, batch_i32, N // BLK, Ww1, bw1, Ww2, bw2,
                Wg1, bg1, Wg2, bg2, Wm1, bm1, Wm2, bm2)
